# baseline (speedup 1.0000x reference)
import functools

import jax
import jax.numpy as jnp
from jax import lax
from jax.experimental import pallas as pl
from jax.experimental.pallas import tpu as pltpu
from jax.experimental.pallas import tpu_sc as plsc

VOCAB = 1000000
EMBED = 32
B = 16384
MAX_SUBWORDS = 10

NC = 2
NS = 16
NW = NC * NS
BPW = B // NW
C = 64
NCHUNK = BPW // C
G = MAX_SUBWORDS
PACK = 128 // EMBED
VTAIL = 999936
NROW = VTAIL // PACK
NB4 = VTAIL // 512
ZROW = C * G
TROW = C * G + 8


def _retile_body(tableT_hbm, tableV_hbm, stage_v, packed_v):
    wid = lax.axis_index("s") * NC + lax.axis_index("c")
    riota = lax.iota(jnp.int32, 16)

    def do_batch(b4, _):
        pltpu.sync_copy(tableT_hbm.at[:, pl.ds(b4 * 512, 512)], stage_v)

        def do_row(r, _):
            for tvl in range(4):
                for h in range(8):
                    v = plsc.load_gather(
                        stage_v,
                        [(h % 2) * 16 + riota,
                         jnp.full((16,), tvl * 128 + r * 4 + (h // 2),
                                  jnp.int32)])
                    packed_v[tvl * 32 + r, pl.ds(h * 16, 16)] = v
            return 0

        lax.fori_loop(0, 32, do_row, 0)
        pltpu.sync_copy(packed_v, tableV_hbm.at[pl.ds(b4 * 128, 128)])
        return 0

    start = wid * 61
    lax.fori_loop(start, start + 61, do_batch, 0)
    @pl.when(wid == 0)
    def _():
        do_batch(NB4 - 1, 0)


def _lookup_body(tableV_hbm, ids_hbm, lens_hbm, tail_hbm, out_hbm,
                 ids_v, idx_v, rows_v, lens_v, out_v, tail_v, sem):
    wid = lax.axis_index("s") * NC + lax.axis_index("c")
    riota = lax.iota(jnp.int32, 16)
    zero16 = jnp.zeros((16,), jnp.float32)
    for h in range(8):
        rows_v[ZROW, pl.ds(h * 16, 16)] = zero16
    pltpu.sync_copy(tail_hbm, tail_v)
    for rt in range(16):
        for h in range(8):
            v = plsc.load_gather(
                tail_v,
                [(h % 2) * 16 + riota,
                 jnp.full((16,), rt * 4 + (h // 2), jnp.int32)])
            rows_v[TROW + rt, pl.ds(h * 16, 16)] = v

    def chunk_body(chunk, _):
        wbase = wid * BPW + chunk * C
        pltpu.sync_copy(ids_hbm.at[wid * NCHUNK + chunk], ids_v)
        pltpu.sync_copy(lens_hbm.at[pl.ds(wbase, C)], lens_v)
        for g in range(G):
            for k in range(C // 16):
                idx_v[g, pl.ds(k * 16, 16)] = jnp.minimum(
                    ids_v[g, pl.ds(k * 16, 16)] >> 2, NROW - 1)
        copies = [
            pltpu.async_copy(tableV_hbm.at[idx_v.at[g]],
                             rows_v.at[pl.ds(g * C, C)], sem)
            for g in range(G)
        ]
        for cp in copies:
            cp.wait()

        def group_body(k, _):
            kb = k * 16
            lens16 = lens_v[pl.ds(kb, 16)]
            linv16 = 1.0 / lens16.astype(jnp.float32)
            acc = [zero16] * EMBED
            for j in range(G):
                ids16 = ids_v[j, pl.ds(kb, 16)]
                off16 = (ids16 & (PACK - 1)) * EMBED
                slotn = j * C + kb + riota
                slott = TROW + ((ids16 - VTAIL) >> 2)
                rowj = jnp.where(
                    lens16 > j,
                    jnp.where(ids16 >= VTAIL, slott, slotn), ZROW)
                for e in range(EMBED):
                    v = plsc.load_gather(rows_v, [rowj, off16 + e])
                    acc[e] = acc[e] + v
            widx = kb + riota
            for e in range(EMBED):
                plsc.store_scatter(out_v, [widx, jnp.full((16,), e, jnp.int32)],
                                   acc[e] * linv16)
            return 0

        lax.fori_loop(0, C // 16, group_body, 0)
        pltpu.sync_copy(out_v, out_hbm.at[pl.ds(wbase, C)])
        return 0

    lax.fori_loop(0, NCHUNK, chunk_body, 0)


@functools.partial(jax.jit, static_argnames=())
def kernel(subword_ids, lengths, table):
    tableT = table.T
    tail = lax.slice(tableT, (0, VTAIL), (EMBED, VOCAB))
    ids3d = subword_ids.reshape(NW * NCHUNK, C, G).transpose(0, 2, 1)
    mesh = plsc.VectorSubcoreMesh(core_axis_name="c", subcore_axis_name="s")
    retile = pl.kernel(
        _retile_body,
        mesh=mesh,
        out_type=jax.ShapeDtypeStruct((NROW, 128), jnp.float32),
        scratch_types=[
            pltpu.VMEM((EMBED, 512), jnp.float32),
            pltpu.VMEM((128, 128), jnp.float32),
        ],
        compiler_params=pltpu.CompilerParams(needs_layout_passes=False),
    )
    tableV = retile(tableT)
    lookup = pl.kernel(
        _lookup_body,
        mesh=mesh,
        out_type=jax.ShapeDtypeStruct((B, EMBED), jnp.float32),
        scratch_types=[
            pltpu.VMEM((G, C), jnp.int32),
            pltpu.VMEM((G, C), jnp.int32),
            pltpu.VMEM((C * G + 24, 128), jnp.float32),
            pltpu.VMEM((C,), jnp.int32),
            pltpu.VMEM((C, EMBED), jnp.float32),
            pltpu.VMEM((EMBED, 64), jnp.float32),
            pltpu.SemaphoreType.DMA,
        ],
        compiler_params=pltpu.CompilerParams(needs_layout_passes=False),
    )
    return lookup(tableV, ids3d, lengths, tail)


# speedup vs baseline: 3.3991x; 3.3991x over previous
import functools

import jax
import jax.numpy as jnp
from jax import lax
from jax.experimental import pallas as pl
from jax.experimental.pallas import tpu as pltpu
from jax.experimental.pallas import tpu_sc as plsc

VOCAB = 1000000
EMBED = 32
B = 16384
MAX_SUBWORDS = 10

NC = 2
NS = 16
NW = NC * NS
BPW = B // NW
C = 64
NCHUNK = BPW // C
G = MAX_SUBWORDS
PACK = 128 // EMBED
VTAIL = 999936
NROW = VTAIL // PACK
NB4 = VTAIL // 512
NITER = 62
ZROW = C * G
TROW = C * G + 8


def _retile_body(tableT_hbm, tableV_hbm, stage_v, packed_v, isem0, isem1,
                 osem0, osem1):
    wid = lax.axis_index("s") * NC + lax.axis_index("c")
    riota = lax.iota(jnp.int32, 16)
    isems = (isem0, isem1)
    osems = (osem0, osem1)
    rowl_base = riota >> 2
    colb = (riota & 3) * 32

    def in_copy(i, s, fire=True):
        mk = pltpu.async_copy if fire else pltpu.make_async_copy
        return mk(
            tableT_hbm.at[:, pl.ds(pl.multiple_of((i * NW + wid) * 512, 512),
                                   512)],
            stage_v.at[s], isems[s])

    def out_copy(i, s, fire=True):
        mk = pltpu.async_copy if fire else pltpu.make_async_copy
        return mk(
            packed_v.at[s], tableV_hbm.at[pl.ds((i * NW + wid) * 128, 128)],
            osems[s])

    in_copy(0, 0)
    in_copy(1, 1)

    def pair_body(p, _):
        for s in range(2):
            i = p * 2 + s
            live = (i * NW + wid) < NB4
            @pl.when(live)
            def _(i=i, s=s):
                in_copy(i, s, fire=False).wait()

                @pl.when(i >= 2)
                def _():
                    out_copy(i - 2, s, fire=False).wait()

                def do_q(q, _):
                    rowl = q * 4 + rowl_base
                    for e in range(EMBED):
                        v = stage_v[s, e, pl.ds(q * 16, 16)]
                        plsc.store_scatter(
                            packed_v.at[s],
                            [rowl, colb + (riota ^ e)], v)
                    return 0

                lax.fori_loop(0, 32, do_q, 0)
                out_copy(i, s)
                @pl.when((i + 2) * NW + wid < NB4)
                def _():
                    in_copy(i + 2, s)

        return 0

    lax.fori_loop(0, NITER // 2, pair_body, 0)
    for s in range(2):
        i = NITER - 2 + s
        @pl.when(i * NW + wid < NB4)
        def _(i=i, s=s):
            out_copy(i, s, fire=False).wait()


def _lookup_body(tableV_hbm, ids_hbm, lens_hbm, tail_hbm, outT_hbm,
                 ids_v, idx_v, rows_v, lens_v, outT_v, tail_v, sem):
    wid = lax.axis_index("s") * NC + lax.axis_index("c")
    riota = lax.iota(jnp.int32, 16)
    zero16 = jnp.zeros((16,), jnp.float32)
    for h in range(8):
        rows_v[ZROW, pl.ds(h * 16, 16)] = zero16
    pltpu.sync_copy(tail_hbm, tail_v)
    for rt in range(16):
        for h in range(8):
            e0 = (h % 2) * 16
            vsub = h // 2
            v = plsc.load_gather(
                tail_v,
                [e0 + riota, jnp.full((16,), rt * 4 + vsub, jnp.int32)])
            cb = 4 * (rt & 3) + vsub
            plsc.store_scatter(
                rows_v.at[TROW + rt],
                [jnp.full((16,), vsub * 32, jnp.int32) + ((e0 + riota) ^ cb)],
                v)

    def chunk_body(chunk, _):
        wbase = wid * BPW + chunk * C
        ob = (chunk & 1) * C
        pltpu.sync_copy(ids_hbm.at[wid * NCHUNK + chunk], ids_v)
        pltpu.sync_copy(lens_hbm.at[pl.ds(wbase, C)], lens_v)
        for g in range(G):
            for k in range(C // 16):
                idx_v[g, pl.ds(k * 16, 16)] = jnp.minimum(
                    ids_v[g, pl.ds(k * 16, 16)] >> 2, NROW - 1)
        copies = [
            pltpu.async_copy(tableV_hbm.at[idx_v.at[g]],
                             rows_v.at[pl.ds(g * C, C)], sem)
            for g in range(G)
        ]
        for cp in copies:
            cp.wait()

        def group_body(k, _):
            kb = k * 16
            lens16 = lens_v[pl.ds(kb, 16)]
            linv16 = 1.0 / lens16.astype(jnp.float32)
            acc = [zero16] * EMBED
            for j in range(G):
                ids16 = ids_v[j, pl.ds(kb, 16)]
                vsub = ids16 & (PACK - 1)
                cb = ((ids16 >> 2) & 3) * 4 + vsub
                colb = vsub * 32
                slotn = j * C + kb + riota
                slott = TROW + ((ids16 - VTAIL) >> 2)
                rowj = jnp.where(
                    lens16 > j,
                    jnp.where(ids16 >= VTAIL, slott, slotn), ZROW)
                for e in range(EMBED):
                    v = plsc.load_gather(rows_v, [rowj, colb + (cb ^ e)])
                    acc[e] = acc[e] + v
            for e in range(EMBED):
                outT_v[e, pl.ds(ob + kb, 16)] = acc[e] * linv16
            return 0

        lax.fori_loop(0, C // 16, group_body, 0)

        @pl.when(chunk & 1 == 1)
        def _():
            pltpu.sync_copy(
                outT_v,
                outT_hbm.at[:, pl.ds(
                    pl.multiple_of(wid * BPW + (chunk - 1) * C, 2 * C),
                    2 * C)])
        return 0

    lax.fori_loop(0, NCHUNK, chunk_body, 0)


@functools.partial(jax.jit, static_argnames=())
def kernel(subword_ids, lengths, table):
    tableT = table.T
    tail = lax.slice(tableT, (0, VTAIL), (EMBED, VOCAB))
    ids3d = subword_ids.reshape(NW * NCHUNK, C, G).transpose(0, 2, 1)
    mesh = plsc.VectorSubcoreMesh(core_axis_name="c", subcore_axis_name="s")
    retile = pl.kernel(
        _retile_body,
        mesh=mesh,
        out_type=jax.ShapeDtypeStruct((NROW, 128), jnp.float32),
        scratch_types=[
            pltpu.VMEM((2, EMBED, 512), jnp.float32),
            pltpu.VMEM((2, 128, 128), jnp.float32),
            pltpu.SemaphoreType.DMA,
            pltpu.SemaphoreType.DMA,
            pltpu.SemaphoreType.DMA,
            pltpu.SemaphoreType.DMA,
        ],
        compiler_params=pltpu.CompilerParams(needs_layout_passes=False),
    )
    tableV = retile(tableT)
    lookup = pl.kernel(
        _lookup_body,
        mesh=mesh,
        out_type=jax.ShapeDtypeStruct((EMBED, B), jnp.float32),
        scratch_types=[
            pltpu.VMEM((G, C), jnp.int32),
            pltpu.VMEM((G, C), jnp.int32),
            pltpu.VMEM((C * G + 24, 128), jnp.float32),
            pltpu.VMEM((C,), jnp.int32),
            pltpu.VMEM((EMBED, 2 * C), jnp.float32),
            pltpu.VMEM((EMBED, 64), jnp.float32),
            pltpu.SemaphoreType.DMA,
        ],
        compiler_params=pltpu.CompilerParams(needs_layout_passes=False),
    )
    outT = lookup(tableV, ids3d, lengths, tail)
    return outT.T


# speedup vs baseline: 3.4090x; 1.0029x over previous
import functools

import jax
import jax.numpy as jnp
from jax import lax
from jax.experimental import pallas as pl
from jax.experimental.pallas import tpu as pltpu
from jax.experimental.pallas import tpu_sc as plsc

VOCAB = 1000000
EMBED = 32
B = 16384
MAX_SUBWORDS = 10

NC = 2
NS = 16
NW = NC * NS
BPW = B // NW
C = 64
NCHUNK = BPW // C
G = MAX_SUBWORDS
PACK = 128 // EMBED
VTAIL = 999936
NROW = VTAIL // PACK
NB4 = VTAIL // 512
NITER = 62
ZROW = C * G
TROW = C * G + 8


def _retile_body(tableT_hbm, tableV_hbm, stage_v, packed_v, isem0, isem1,
                 osem0, osem1):
    wid = lax.axis_index("s") * NC + lax.axis_index("c")
    riota = lax.iota(jnp.int32, 16)
    isems = (isem0, isem1)
    osems = (osem0, osem1)
    rowl_base = riota >> 2
    colb = (riota & 3) * 32
    cols = [colb + (riota ^ e) for e in range(EMBED)]

    def in_copy(i, s, fire=True):
        mk = pltpu.async_copy if fire else pltpu.make_async_copy
        return mk(
            tableT_hbm.at[:, pl.ds(pl.multiple_of((i * NW + wid) * 512, 512),
                                   512)],
            stage_v.at[s], isems[s])

    def out_copy(i, s, fire=True):
        mk = pltpu.async_copy if fire else pltpu.make_async_copy
        return mk(
            packed_v.at[s], tableV_hbm.at[pl.ds((i * NW + wid) * 128, 128)],
            osems[s])

    in_copy(0, 0)
    in_copy(1, 1)

    def pair_body(p, _):
        for s in range(2):
            i = p * 2 + s
            live = (i * NW + wid) < NB4
            @pl.when(live)
            def _(i=i, s=s):
                in_copy(i, s, fire=False).wait()

                @pl.when(i >= 2)
                def _():
                    out_copy(i - 2, s, fire=False).wait()

                def do_q8(q8, _, s=s):
                    for qs in range(8):
                        q = q8 * 8 + qs
                        rowl = q * 4 + rowl_base
                        for e in range(EMBED):
                            v = stage_v[s, e, pl.ds(q * 16, 16)]
                            plsc.store_scatter(
                                packed_v.at[s], [rowl, cols[e]], v)
                    return 0

                lax.fori_loop(0, 4, do_q8, 0)
                out_copy(i, s)
                @pl.when((i + 2) * NW + wid < NB4)
                def _():
                    in_copy(i + 2, s)

        return 0

    lax.fori_loop(0, NITER // 2, pair_body, 0)
    for s in range(2):
        i = NITER - 2 + s
        @pl.when(i * NW + wid < NB4)
        def _(i=i, s=s):
            out_copy(i, s, fire=False).wait()


def _lookup_body(tableV_hbm, ids_hbm, lens_hbm, tail_hbm, outT_hbm,
                 ids_v, idx_v, rows_v, lens_v, outT_v, tail_v, sem):
    wid = lax.axis_index("s") * NC + lax.axis_index("c")
    riota = lax.iota(jnp.int32, 16)
    zero16 = jnp.zeros((16,), jnp.float32)
    for h in range(8):
        rows_v[ZROW, pl.ds(h * 16, 16)] = zero16
    pltpu.sync_copy(tail_hbm, tail_v)
    for rt in range(16):
        for h in range(8):
            e0 = (h % 2) * 16
            vsub = h // 2
            v = plsc.load_gather(
                tail_v,
                [e0 + riota, jnp.full((16,), rt * 4 + vsub, jnp.int32)])
            cb = 4 * (rt & 3) + vsub
            plsc.store_scatter(
                rows_v.at[TROW + rt],
                [jnp.full((16,), vsub * 32, jnp.int32) + ((e0 + riota) ^ cb)],
                v)

    def chunk_body(chunk, _):
        wbase = wid * BPW + chunk * C
        ob = (chunk & 1) * C
        pltpu.sync_copy(ids_hbm.at[wid * NCHUNK + chunk], ids_v)
        pltpu.sync_copy(lens_hbm.at[pl.ds(wbase, C)], lens_v)
        for g in range(G):
            for k in range(C // 16):
                idx_v[g, pl.ds(k * 16, 16)] = jnp.minimum(
                    ids_v[g, pl.ds(k * 16, 16)] >> 2, NROW - 1)
        copies = [
            pltpu.async_copy(tableV_hbm.at[idx_v.at[g]],
                             rows_v.at[pl.ds(g * C, C)], sem)
            for g in range(G)
        ]
        for cp in copies:
            cp.wait()

        def group_body(k, _):
            kb = k * 16
            lens16 = lens_v[pl.ds(kb, 16)]
            linv16 = 1.0 / lens16.astype(jnp.float32)
            acc = [zero16] * EMBED
            for j in range(G):
                ids16 = ids_v[j, pl.ds(kb, 16)]
                vsub = ids16 & (PACK - 1)
                cb = ((ids16 >> 2) & 3) * 4 + vsub
                colb = vsub * 32
                slotn = j * C + kb + riota
                slott = TROW + ((ids16 - VTAIL) >> 2)
                rowj = jnp.where(
                    lens16 > j,
                    jnp.where(ids16 >= VTAIL, slott, slotn), ZROW)
                for e in range(EMBED):
                    v = plsc.load_gather(rows_v, [rowj, colb + (cb ^ e)])
                    acc[e] = acc[e] + v
            for e in range(EMBED):
                outT_v[e, pl.ds(ob + kb, 16)] = acc[e] * linv16
            return 0

        lax.fori_loop(0, C // 16, group_body, 0)

        @pl.when(chunk & 1 == 1)
        def _():
            pltpu.sync_copy(
                outT_v,
                outT_hbm.at[:, pl.ds(
                    pl.multiple_of(wid * BPW + (chunk - 1) * C, 2 * C),
                    2 * C)])
        return 0

    lax.fori_loop(0, NCHUNK, chunk_body, 0)


@functools.partial(jax.jit, static_argnames=())
def kernel(subword_ids, lengths, table):
    tableT = table.T
    tail = lax.slice(tableT, (0, VTAIL), (EMBED, VOCAB))
    ids3d = subword_ids.reshape(NW * NCHUNK, C, G).transpose(0, 2, 1)
    mesh = plsc.VectorSubcoreMesh(core_axis_name="c", subcore_axis_name="s")
    retile = pl.kernel(
        _retile_body,
        mesh=mesh,
        out_type=jax.ShapeDtypeStruct((NROW, 128), jnp.float32),
        scratch_types=[
            pltpu.VMEM((2, EMBED, 512), jnp.float32),
            pltpu.VMEM((2, 128, 128), jnp.float32),
            pltpu.SemaphoreType.DMA,
            pltpu.SemaphoreType.DMA,
            pltpu.SemaphoreType.DMA,
            pltpu.SemaphoreType.DMA,
        ],
        compiler_params=pltpu.CompilerParams(needs_layout_passes=False),
    )
    tableV = retile(tableT)
    lookup = pl.kernel(
        _lookup_body,
        mesh=mesh,
        out_type=jax.ShapeDtypeStruct((EMBED, B), jnp.float32),
        scratch_types=[
            pltpu.VMEM((G, C), jnp.int32),
            pltpu.VMEM((G, C), jnp.int32),
            pltpu.VMEM((C * G + 24, 128), jnp.float32),
            pltpu.VMEM((C,), jnp.int32),
            pltpu.VMEM((EMBED, 2 * C), jnp.float32),
            pltpu.VMEM((EMBED, 64), jnp.float32),
            pltpu.SemaphoreType.DMA,
        ],
        compiler_params=pltpu.CompilerParams(needs_layout_passes=False),
    )
    outT = lookup(tableV, ids3d, lengths, tail)
    return outT.T


# speedup vs baseline: 5.6995x; 1.6719x over previous
import functools

import jax
import jax.numpy as jnp
from jax import lax
from jax.experimental import pallas as pl
from jax.experimental.pallas import tpu as pltpu
from jax.experimental.pallas import tpu_sc as plsc

VOCAB = 1000000
EMBED = 32
B = 16384
MAX_SUBWORDS = 10

NC = 2
NS = 16
NW = NC * NS
BPW = B // NW
C = 64
NCHUNK = BPW // C
G = MAX_SUBWORDS
PACK = 128 // EMBED
VTAIL = 999936
NROW = VTAIL // PACK
NB4 = VTAIL // 512
NITER = 62
ZROW = C * G
TROW = C * G + 8


def _retile_body(tableT_hbm, tableV_hbm, stage_v, packed_v, isem0, isem1,
                 osem0, osem1):
    wid = lax.axis_index("s") * NC + lax.axis_index("c")
    riota = lax.iota(jnp.int32, 16)
    isems = (isem0, isem1)
    osems = (osem0, osem1)
    rowl_base = riota >> 2
    colb = (riota & 3) * 32
    cols = [colb + (riota ^ e) for e in range(EMBED)]

    def in_copy(i, s, fire=True):
        mk = pltpu.async_copy if fire else pltpu.make_async_copy
        return mk(
            tableT_hbm.at[:, pl.ds(pl.multiple_of((i * NW + wid) * 512, 512),
                                   512)],
            stage_v.at[s], isems[s])

    def out_copy(i, s, fire=True):
        mk = pltpu.async_copy if fire else pltpu.make_async_copy
        return mk(
            packed_v.at[s], tableV_hbm.at[pl.ds((i * NW + wid) * 128, 128)],
            osems[s])

    in_copy(0, 0)
    in_copy(1, 1)

    def pair_body(p, _):
        for s in range(2):
            i = p * 2 + s
            live = (i * NW + wid) < NB4
            @pl.when(live)
            def _(i=i, s=s):
                in_copy(i, s, fire=False).wait()

                @pl.when(i >= 2)
                def _():
                    out_copy(i - 2, s, fire=False).wait()

                def do_q8(q8, _, s=s):
                    for qs in range(8):
                        q = q8 * 8 + qs
                        rowl = q * 4 + rowl_base
                        for eg in range(EMBED // 8):
                            vs = [stage_v[s, eg * 8 + i, pl.ds(q * 16, 16)]
                                  for i in range(8)]
                            for i in range(8):
                                plsc.store_scatter(
                                    packed_v.at[s],
                                    [rowl, cols[eg * 8 + i]], vs[i])
                    return 0

                lax.fori_loop(0, 4, do_q8, 0)
                out_copy(i, s)
                @pl.when((i + 2) * NW + wid < NB4)
                def _():
                    in_copy(i + 2, s)

        return 0

    lax.fori_loop(0, NITER // 2, pair_body, 0)
    for s in range(2):
        i = NITER - 2 + s
        @pl.when(i * NW + wid < NB4)
        def _(i=i, s=s):
            out_copy(i, s, fire=False).wait()


def _lookup_body(tableV_hbm, ids_hbm, lens_hbm, tail_hbm, outT_hbm,
                 ids_v, idx_v, rows_v, lens_v, outT_v, tail_v, sem):
    wid = lax.axis_index("s") * NC + lax.axis_index("c")
    riota = lax.iota(jnp.int32, 16)
    zero16 = jnp.zeros((16,), jnp.float32)
    for h in range(8):
        rows_v[ZROW, pl.ds(h * 16, 16)] = zero16
    pltpu.sync_copy(tail_hbm, tail_v)
    for rt in range(16):
        for h in range(8):
            e0 = (h % 2) * 16
            vsub = h // 2
            v = plsc.load_gather(
                tail_v,
                [e0 + riota, jnp.full((16,), rt * 4 + vsub, jnp.int32)])
            cb = 4 * (rt & 3) + vsub
            plsc.store_scatter(
                rows_v.at[TROW + rt],
                [jnp.full((16,), vsub * 32, jnp.int32) + ((e0 + riota) ^ cb)],
                v)

    def chunk_body(chunk, _):
        wbase = wid * BPW + chunk * C
        ob = (chunk & 1) * C
        pltpu.sync_copy(ids_hbm.at[wid * NCHUNK + chunk], ids_v)
        pltpu.sync_copy(lens_hbm.at[pl.ds(wbase, C)], lens_v)
        for g in range(G):
            for k in range(C // 16):
                idx_v[g, pl.ds(k * 16, 16)] = jnp.minimum(
                    ids_v[g, pl.ds(k * 16, 16)] >> 2, NROW - 1)
        copies = [
            pltpu.async_copy(tableV_hbm.at[idx_v.at[g]],
                             rows_v.at[pl.ds(g * C, C)], sem)
            for g in range(G)
        ]
        for cp in copies:
            cp.wait()

        def group_body(k, _):
            kb = k * 16
            lens16 = lens_v[pl.ds(kb, 16)]
            linv16 = 1.0 / lens16.astype(jnp.float32)
            acc = [zero16] * EMBED
            for j in range(G):
                ids16 = ids_v[j, pl.ds(kb, 16)]
                vsub = ids16 & (PACK - 1)
                cb = ((ids16 >> 2) & 3) * 4 + vsub
                colb = vsub * 32
                slotn = j * C + kb + riota
                slott = TROW + ((ids16 - VTAIL) >> 2)
                rowj = jnp.where(
                    lens16 > j,
                    jnp.where(ids16 >= VTAIL, slott, slotn), ZROW)
                for e in range(EMBED):
                    v = plsc.load_gather(rows_v, [rowj, colb + (cb ^ e)])
                    acc[e] = acc[e] + v
            for e in range(EMBED):
                outT_v[e, pl.ds(ob + kb, 16)] = acc[e] * linv16
            return 0

        lax.fori_loop(0, C // 16, group_body, 0)

        @pl.when(chunk & 1 == 1)
        def _():
            pltpu.sync_copy(
                outT_v,
                outT_hbm.at[:, pl.ds(
                    pl.multiple_of(wid * BPW + (chunk - 1) * C, 2 * C),
                    2 * C)])
        return 0

    lax.fori_loop(0, NCHUNK, chunk_body, 0)


@functools.partial(jax.jit, static_argnames=())
def kernel(subword_ids, lengths, table):
    tableT = table.T
    tail = lax.slice(tableT, (0, VTAIL), (EMBED, VOCAB))
    ids3d = subword_ids.reshape(NW * NCHUNK, C, G).transpose(0, 2, 1)
    mesh = plsc.VectorSubcoreMesh(core_axis_name="c", subcore_axis_name="s")
    retile = pl.kernel(
        _retile_body,
        mesh=mesh,
        out_type=jax.ShapeDtypeStruct((NROW, 128), jnp.float32),
        scratch_types=[
            pltpu.VMEM((2, EMBED, 512), jnp.float32),
            pltpu.VMEM((2, 128, 128), jnp.float32),
            pltpu.SemaphoreType.DMA,
            pltpu.SemaphoreType.DMA,
            pltpu.SemaphoreType.DMA,
            pltpu.SemaphoreType.DMA,
        ],
        compiler_params=pltpu.CompilerParams(needs_layout_passes=False),
    )
    tableV = retile(tableT)
    lookup = pl.kernel(
        _lookup_body,
        mesh=mesh,
        out_type=jax.ShapeDtypeStruct((EMBED, B), jnp.float32),
        scratch_types=[
            pltpu.VMEM((G, C), jnp.int32),
            pltpu.VMEM((G, C), jnp.int32),
            pltpu.VMEM((C * G + 24, 128), jnp.float32),
            pltpu.VMEM((C,), jnp.int32),
            pltpu.VMEM((EMBED, 2 * C), jnp.float32),
            pltpu.VMEM((EMBED, 64), jnp.float32),
            pltpu.SemaphoreType.DMA,
        ],
        compiler_params=pltpu.CompilerParams(needs_layout_passes=False),
    )
    outT = lookup(tableV, ids3d, lengths, tail)
    return outT.T


# speedup vs baseline: 6.0431x; 1.0603x over previous
import functools

import jax
import jax.numpy as jnp
from jax import lax
from jax.experimental import pallas as pl
from jax.experimental.pallas import tpu as pltpu
from jax.experimental.pallas import tpu_sc as plsc

VOCAB = 1000000
EMBED = 32
B = 16384
MAX_SUBWORDS = 10

NC = 2
NS = 16
NW = NC * NS
BPW = B // NW
C = 32
CSUP = 128
NCHUNK = BPW // C
G = MAX_SUBWORDS
PACK = 128 // EMBED
VTAIL = 999936
NROW = VTAIL // PACK
NB4 = VTAIL // 512
NITER = 62
ZROW = C * G
TROW = C * G + 8
SBUF = C * G + 24


def _retile_body(tableT_hbm, tableV_hbm, stage_v, packed_v, isem0, isem1,
                 osem0, osem1):
    wid = lax.axis_index("s") * NC + lax.axis_index("c")
    riota = lax.iota(jnp.int32, 16)
    isems = (isem0, isem1)
    osems = (osem0, osem1)
    rowl_base = riota >> 2
    colb = (riota & 3) * 32
    cols = [colb + (riota ^ e) for e in range(EMBED)]

    def in_copy(i, s, fire=True):
        mk = pltpu.async_copy if fire else pltpu.make_async_copy
        return mk(
            tableT_hbm.at[:, pl.ds(pl.multiple_of((i * NW + wid) * 512, 512),
                                   512)],
            stage_v.at[s], isems[s])

    def out_copy(i, s, fire=True):
        mk = pltpu.async_copy if fire else pltpu.make_async_copy
        return mk(
            packed_v.at[s], tableV_hbm.at[pl.ds((i * NW + wid) * 128, 128)],
            osems[s])

    in_copy(0, 0)
    in_copy(1, 1)

    def pair_body(p, _):
        for s in range(2):
            i = p * 2 + s
            live = (i * NW + wid) < NB4
            @pl.when(live)
            def _(i=i, s=s):
                in_copy(i, s, fire=False).wait()

                @pl.when(i >= 2)
                def _():
                    out_copy(i - 2, s, fire=False).wait()

                def do_q8(q8, _, s=s):
                    for qs in range(8):
                        q = q8 * 8 + qs
                        rowl = q * 4 + rowl_base
                        for eg in range(EMBED // 8):
                            vs = [stage_v[s, eg * 8 + i, pl.ds(q * 16, 16)]
                                  for i in range(8)]
                            for i in range(8):
                                plsc.store_scatter(
                                    packed_v.at[s],
                                    [rowl, cols[eg * 8 + i]], vs[i])
                    return 0

                lax.fori_loop(0, 4, do_q8, 0)
                out_copy(i, s)
                @pl.when((i + 2) * NW + wid < NB4)
                def _():
                    in_copy(i + 2, s)

        return 0

    lax.fori_loop(0, NITER // 2, pair_body, 0)
    for s in range(2):
        i = NITER - 2 + s
        @pl.when(i * NW + wid < NB4)
        def _(i=i, s=s):
            out_copy(i, s, fire=False).wait()


def _lookup_body(tableV_hbm, idsT_hbm, lens_hbm, tail_hbm, outT_hbm,
                 ids2_v, idx_v, rows_v, lens_v, outT_v, tail_v, sem0):
    wid = lax.axis_index("s") * NC + lax.axis_index("c")
    riota = lax.iota(jnp.int32, 16)
    zero16 = jnp.zeros((16,), jnp.float32)
    for s in range(2):
        for h in range(8):
            rows_v[s * SBUF + ZROW, pl.ds(h * 16, 16)] = zero16
    pltpu.sync_copy(tail_hbm, tail_v)
    for rt in range(16):
        for h in range(8):
            e0 = (h % 2) * 16
            vsub = h // 2
            v = plsc.load_gather(
                tail_v,
                [e0 + riota, jnp.full((16,), rt * 4 + vsub, jnp.int32)])
            cb = 4 * (rt & 3) + vsub
            for s in range(2):
                plsc.store_scatter(
                    rows_v.at[s * SBUF + TROW + rt],
                    [jnp.full((16,), vsub * 32, jnp.int32)
                     + ((e0 + riota) ^ cb)], v)

    def load_sup(c):
        sup = c // (CSUP // C)
        base = pl.multiple_of(wid * BPW + sup * CSUP, CSUP)
        for p in range(2):
            @pl.when(sup & 1 == p)
            def _(p=p):
                pltpu.sync_copy(idsT_hbm.at[wid * (BPW // CSUP) + sup],
                                ids2_v.at[pl.ds(p * G, G)])
                pltpu.sync_copy(lens_hbm.at[pl.ds(base, CSUP)],
                                lens_v.at[pl.ds(p * CSUP, CSUP)])

    def prep_and_fire(c, s):
        p = (c // (CSUP // C)) & 1
        cb32 = (c % (CSUP // C)) * C
        for g in range(G):
            for k in range(C // 16):
                idx_v[s * G + g, pl.ds(k * 16, 16)] = jnp.minimum(
                    ids2_v[p * G + g, pl.ds(cb32 + k * 16, 16)] >> 2, NROW - 1)
        for g in range(G):
            pltpu.async_copy(tableV_hbm.at[idx_v.at[s * G + g]],
                             rows_v.at[pl.ds(s * SBUF + g * C, C)], sem0)

    def drain(c, s):
        for g in range(G):
            pltpu.make_async_copy(
                tableV_hbm.at[pl.ds(0, C)],
                rows_v.at[pl.ds(s * SBUF + g * C, C)], sem0).wait()

    def compute(c, s):
        p = (c // (CSUP // C)) & 1
        cb32 = (c % (CSUP // C)) * C

        def group_body(k, _):
            kb = k * 16
            lens16 = lens_v[pl.ds(p * CSUP + cb32 + kb, 16)]
            linv16 = 1.0 / lens16.astype(jnp.float32)
            acc = [zero16] * EMBED
            for j in range(G):
                ids16 = ids2_v[p * G + j, pl.ds(cb32 + kb, 16)]
                vsub = ids16 & (PACK - 1)
                cb = ((ids16 >> 2) & 3) * 4 + vsub
                colb = vsub * 32
                slotn = s * SBUF + j * C + kb + riota
                slott = s * SBUF + TROW + ((ids16 - VTAIL) >> 2)
                rowj = jnp.where(
                    lens16 > j,
                    jnp.where(ids16 >= VTAIL, slott, slotn),
                    s * SBUF + ZROW)
                for e in range(EMBED):
                    v = plsc.load_gather(rows_v, [rowj, colb + (cb ^ e)])
                    acc[e] = acc[e] + v
            for e in range(EMBED):
                outT_v[e, pl.ds(cb32 + kb, 16)] = acc[e] * linv16
            return 0

        lax.fori_loop(0, C // 16, group_body, 0)

    load_sup(0)
    prep_and_fire(0, 0)

    def pair_body(pi, _):
        for s in range(2):
            c = pi * 2 + s
            c1 = c + 1

            @pl.when(jnp.logical_and(c1 < NCHUNK, c1 % (CSUP // C) == 0))
            def _(c1=c1):
                load_sup(c1)

            drain(c, s)

            @pl.when(c1 < NCHUNK)
            def _(c1=c1, s=s):
                prep_and_fire(c1, 1 - s)
            compute(c, s)

            @pl.when(c % (CSUP // C) == CSUP // C - 1)
            def _(c=c):
                sup = c // (CSUP // C)
                pltpu.sync_copy(
                    outT_v,
                    outT_hbm.at[:, pl.ds(
                        pl.multiple_of(wid * BPW + sup * CSUP, CSUP), CSUP)])
        return 0

    lax.fori_loop(0, NCHUNK // 2, pair_body, 0)


@functools.partial(jax.jit, static_argnames=())
def kernel(subword_ids, lengths, table):
    tableT = table.T
    tail = lax.slice(tableT, (0, VTAIL), (EMBED, VOCAB))
    idsT = subword_ids.reshape(NW * BPW // CSUP, CSUP,
                               G).transpose(0, 2, 1)
    mesh = plsc.VectorSubcoreMesh(core_axis_name="c", subcore_axis_name="s")
    retile = pl.kernel(
        _retile_body,
        mesh=mesh,
        out_type=jax.ShapeDtypeStruct((NROW, 128), jnp.float32),
        scratch_types=[
            pltpu.VMEM((2, EMBED, 512), jnp.float32),
            pltpu.VMEM((2, 128, 128), jnp.float32),
            pltpu.SemaphoreType.DMA,
            pltpu.SemaphoreType.DMA,
            pltpu.SemaphoreType.DMA,
            pltpu.SemaphoreType.DMA,
        ],
        compiler_params=pltpu.CompilerParams(needs_layout_passes=False),
    )
    tableV = retile(tableT)
    lookup = pl.kernel(
        _lookup_body,
        mesh=mesh,
        out_type=jax.ShapeDtypeStruct((EMBED, B), jnp.float32),
        scratch_types=[
            pltpu.VMEM((2 * G, CSUP), jnp.int32),
            pltpu.VMEM((2 * G, C), jnp.int32),
            pltpu.VMEM((2 * SBUF, 128), jnp.float32),
            pltpu.VMEM((2 * CSUP,), jnp.int32),
            pltpu.VMEM((EMBED, CSUP), jnp.float32),
            pltpu.VMEM((EMBED, 64), jnp.float32),
            pltpu.SemaphoreType.DMA,
        ],
        compiler_params=pltpu.CompilerParams(needs_layout_passes=False),
    )
    outT = lookup(tableV, idsT, lengths, tail)
    return outT.T


# speedup vs baseline: 6.3100x; 1.0442x over previous
import functools

import jax
import jax.numpy as jnp
from jax import lax
from jax.experimental import pallas as pl
from jax.experimental.pallas import tpu as pltpu
from jax.experimental.pallas import tpu_sc as plsc

VOCAB = 1000000
EMBED = 32
B = 16384
MAX_SUBWORDS = 10

NC = 2
NS = 16
NW = NC * NS
BPW = B // NW
C = 32
CSUP = 128
NCHUNK = BPW // C
G = MAX_SUBWORDS
PACK = 128 // EMBED
VTAIL = 999936
NROW = VTAIL // PACK
NB4 = VTAIL // 768
NITER = 42
ZROW = C * G
TROW = C * G + 8
SBUF = C * G + 24


def _retile_body(tableT_hbm, tableV_hbm, stage_v, packed_v, isem0, isem1,
                 osem0, osem1):
    wid = lax.axis_index("s") * NC + lax.axis_index("c")
    riota = lax.iota(jnp.int32, 16)
    isems = (isem0, isem1)
    osems = (osem0, osem1)
    rowl_base = riota >> 2
    colb = (riota & 3) * 32
    cols = [colb + (riota ^ e) for e in range(EMBED)]

    def in_copy(i, s, fire=True):
        mk = pltpu.async_copy if fire else pltpu.make_async_copy
        return mk(
            tableT_hbm.at[:, pl.ds(pl.multiple_of((i * NW + wid) * 768, 768),
                                   768)],
            stage_v.at[s], isems[s])

    def out_copy(i, s, fire=True):
        mk = pltpu.async_copy if fire else pltpu.make_async_copy
        return mk(
            packed_v.at[s], tableV_hbm.at[pl.ds((i * NW + wid) * 192, 192)],
            osems[s])

    in_copy(0, 0)
    in_copy(1, 1)

    def pair_body(p, _):
        for s in range(2):
            i = p * 2 + s
            live = (i * NW + wid) < NB4
            @pl.when(live)
            def _(i=i, s=s):
                in_copy(i, s, fire=False).wait()

                @pl.when(i >= 2)
                def _():
                    out_copy(i - 2, s, fire=False).wait()

                def do_q8(q8, _, s=s):
                    for qs in range(8):
                        q = q8 * 8 + qs
                        rowl = q * 4 + rowl_base
                        for eg in range(EMBED // 8):
                            vs = [stage_v[s, eg * 8 + i, pl.ds(q * 16, 16)]
                                  for i in range(8)]
                            for i in range(8):
                                plsc.store_scatter(
                                    packed_v.at[s],
                                    [rowl, cols[eg * 8 + i]], vs[i])
                    return 0

                lax.fori_loop(0, 6, do_q8, 0)
                out_copy(i, s)
                @pl.when((i + 2) * NW + wid < NB4)
                def _():
                    in_copy(i + 2, s)

        return 0

    lax.fori_loop(0, NITER // 2, pair_body, 0)
    for s in range(2):
        i = NITER - 2 + s
        @pl.when(i * NW + wid < NB4)
        def _(i=i, s=s):
            out_copy(i, s, fire=False).wait()


def _lookup_body(tableV_hbm, idsT_hbm, lens_hbm, tail_hbm, outT_hbm,
                 ids2_v, idx_v, rows_v, lens_v, outT_v, tail_v, sem0):
    wid = lax.axis_index("s") * NC + lax.axis_index("c")
    riota = lax.iota(jnp.int32, 16)
    zero16 = jnp.zeros((16,), jnp.float32)
    for s in range(2):
        for h in range(8):
            rows_v[s * SBUF + ZROW, pl.ds(h * 16, 16)] = zero16
    pltpu.sync_copy(tail_hbm, tail_v)
    for rt in range(16):
        for h in range(8):
            e0 = (h % 2) * 16
            vsub = h // 2
            v = plsc.load_gather(
                tail_v,
                [e0 + riota, jnp.full((16,), rt * 4 + vsub, jnp.int32)])
            cb = 4 * (rt & 3) + vsub
            for s in range(2):
                plsc.store_scatter(
                    rows_v.at[s * SBUF + TROW + rt],
                    [jnp.full((16,), vsub * 32, jnp.int32)
                     + ((e0 + riota) ^ cb)], v)

    def load_sup(c):
        sup = c // (CSUP // C)
        base = pl.multiple_of(wid * BPW + sup * CSUP, CSUP)
        for p in range(2):
            @pl.when(sup & 1 == p)
            def _(p=p):
                pltpu.sync_copy(idsT_hbm.at[wid * (BPW // CSUP) + sup],
                                ids2_v.at[pl.ds(p * G, G)])
                pltpu.sync_copy(lens_hbm.at[pl.ds(base, CSUP)],
                                lens_v.at[pl.ds(p * CSUP, CSUP)])

    def prep_and_fire(c, s):
        p = (c // (CSUP // C)) & 1
        cb32 = (c % (CSUP // C)) * C
        for g in range(G):
            for k in range(C // 16):
                idx_v[s * G + g, pl.ds(k * 16, 16)] = jnp.minimum(
                    ids2_v[p * G + g, pl.ds(cb32 + k * 16, 16)] >> 2, NROW - 1)
        for g in range(G):
            pltpu.async_copy(tableV_hbm.at[idx_v.at[s * G + g]],
                             rows_v.at[pl.ds(s * SBUF + g * C, C)], sem0)

    def drain(c, s):
        for g in range(G):
            pltpu.make_async_copy(
                tableV_hbm.at[pl.ds(0, C)],
                rows_v.at[pl.ds(s * SBUF + g * C, C)], sem0).wait()

    def compute(c, s):
        p = (c // (CSUP // C)) & 1
        cb32 = (c % (CSUP // C)) * C

        def group_body(k, _):
            kb = k * 16
            lens16 = lens_v[pl.ds(p * CSUP + cb32 + kb, 16)]
            linv16 = 1.0 / lens16.astype(jnp.float32)
            acc = [zero16] * EMBED
            for j in range(G):
                ids16 = ids2_v[p * G + j, pl.ds(cb32 + kb, 16)]
                vsub = ids16 & (PACK - 1)
                cb = ((ids16 >> 2) & 3) * 4 + vsub
                colb = vsub * 32
                slotn = s * SBUF + j * C + kb + riota
                slott = s * SBUF + TROW + ((ids16 - VTAIL) >> 2)
                rowj = jnp.where(
                    lens16 > j,
                    jnp.where(ids16 >= VTAIL, slott, slotn),
                    s * SBUF + ZROW)
                for e in range(EMBED):
                    v = plsc.load_gather(rows_v, [rowj, colb + (cb ^ e)])
                    acc[e] = acc[e] + v
            for e in range(EMBED):
                outT_v[e, pl.ds(cb32 + kb, 16)] = acc[e] * linv16
            return 0

        lax.fori_loop(0, C // 16, group_body, 0)

    load_sup(0)
    prep_and_fire(0, 0)

    def pair_body(pi, _):
        for s in range(2):
            c = pi * 2 + s
            c1 = c + 1

            @pl.when(jnp.logical_and(c1 < NCHUNK, c1 % (CSUP // C) == 0))
            def _(c1=c1):
                load_sup(c1)

            drain(c, s)

            @pl.when(c1 < NCHUNK)
            def _(c1=c1, s=s):
                prep_and_fire(c1, 1 - s)
            compute(c, s)

            @pl.when(c % (CSUP // C) == CSUP // C - 1)
            def _(c=c):
                sup = c // (CSUP // C)
                pltpu.sync_copy(
                    outT_v,
                    outT_hbm.at[:, pl.ds(
                        pl.multiple_of(wid * BPW + sup * CSUP, CSUP), CSUP)])
        return 0

    lax.fori_loop(0, NCHUNK // 2, pair_body, 0)


@functools.partial(jax.jit, static_argnames=())
def kernel(subword_ids, lengths, table):
    tableT = table.T
    tail = lax.slice(tableT, (0, VTAIL), (EMBED, VOCAB))
    idsT = subword_ids.reshape(NW * BPW // CSUP, CSUP,
                               G).transpose(0, 2, 1)
    mesh = plsc.VectorSubcoreMesh(core_axis_name="c", subcore_axis_name="s")
    retile = pl.kernel(
        _retile_body,
        mesh=mesh,
        out_type=jax.ShapeDtypeStruct((NROW, 128), jnp.float32),
        scratch_types=[
            pltpu.VMEM((2, EMBED, 768), jnp.float32),
            pltpu.VMEM((2, 192, 128), jnp.float32),
            pltpu.SemaphoreType.DMA,
            pltpu.SemaphoreType.DMA,
            pltpu.SemaphoreType.DMA,
            pltpu.SemaphoreType.DMA,
        ],
        compiler_params=pltpu.CompilerParams(needs_layout_passes=False),
    )
    tableV = retile(tableT)
    lookup = pl.kernel(
        _lookup_body,
        mesh=mesh,
        out_type=jax.ShapeDtypeStruct((EMBED, B), jnp.float32),
        scratch_types=[
            pltpu.VMEM((2 * G, CSUP), jnp.int32),
            pltpu.VMEM((2 * G, C), jnp.int32),
            pltpu.VMEM((2 * SBUF, 128), jnp.float32),
            pltpu.VMEM((2 * CSUP,), jnp.int32),
            pltpu.VMEM((EMBED, CSUP), jnp.float32),
            pltpu.VMEM((EMBED, 64), jnp.float32),
            pltpu.SemaphoreType.DMA,
        ],
        compiler_params=pltpu.CompilerParams(needs_layout_passes=False),
    )
    outT = lookup(tableV, idsT, lengths, tail)
    return outT.T


# speedup vs baseline: 6.3215x; 1.0018x over previous
import functools

import jax
import jax.numpy as jnp
from jax import lax
from jax.experimental import pallas as pl
from jax.experimental.pallas import tpu as pltpu
from jax.experimental.pallas import tpu_sc as plsc

VOCAB = 1000000
EMBED = 32
B = 16384
MAX_SUBWORDS = 10

NC = 2
NS = 16
NW = NC * NS
BPW = B // NW
C = 32
CSUP = 128
NCHUNK = BPW // C
G = MAX_SUBWORDS
PACK = 128 // EMBED
VTAIL = 999936
NROW = VTAIL // PACK
NB4 = VTAIL // 768
NITER = 42
ZROW = C * G
TROW = C * G + 8
SBUF = C * G + 24


def _retile_body(tableT_hbm, tableV_hbm, stage_v, packed_v, isem0, isem1,
                 osem0, osem1):
    wid = lax.axis_index("s") * NC + lax.axis_index("c")
    riota = lax.iota(jnp.int32, 16)
    isems = (isem0, isem1)
    osems = (osem0, osem1)
    rowl_base = riota >> 2
    colb = (riota & 3) * 32
    cols = [colb + (riota ^ e) for e in range(EMBED)]

    def in_copy(i, s, fire=True):
        mk = pltpu.async_copy if fire else pltpu.make_async_copy
        return mk(
            tableT_hbm.at[:, pl.ds(pl.multiple_of((i * NW + wid) * 768, 768),
                                   768)],
            stage_v.at[s], isems[s])

    def out_copy(i, s, fire=True):
        mk = pltpu.async_copy if fire else pltpu.make_async_copy
        return mk(
            packed_v.at[s], tableV_hbm.at[pl.ds((i * NW + wid) * 192, 192)],
            osems[s])

    in_copy(0, 0)
    in_copy(1, 1)

    def pair_body(p, _):
        for s in range(2):
            i = p * 2 + s
            live = (i * NW + wid) < NB4
            @pl.when(live)
            def _(i=i, s=s):
                in_copy(i, s, fire=False).wait()

                @pl.when(i >= 2)
                def _():
                    out_copy(i - 2, s, fire=False).wait()

                def do_q8(q8, _, s=s):
                    for qs in range(8):
                        q = q8 * 8 + qs
                        rowl = q * 4 + rowl_base
                        for eg in range(EMBED // 8):
                            vs = [stage_v[s, eg * 8 + i, pl.ds(q * 16, 16)]
                                  for i in range(8)]
                            for i in range(8):
                                plsc.store_scatter(
                                    packed_v.at[s],
                                    [rowl, cols[eg * 8 + i]], vs[i])
                    return 0

                lax.fori_loop(0, 6, do_q8, 0)
                out_copy(i, s)
                @pl.when((i + 2) * NW + wid < NB4)
                def _():
                    in_copy(i + 2, s)

        return 0

    lax.fori_loop(0, NITER // 2, pair_body, 0)
    for s in range(2):
        i = NITER - 2 + s
        @pl.when(i * NW + wid < NB4)
        def _(i=i, s=s):
            out_copy(i, s, fire=False).wait()


def _lookup_body(tableV_hbm, idsT_hbm, lens_hbm, tail_hbm, outT_hbm,
                 ids2_v, idx_v, rows_v, lens_v, outT_v, tail_v, sem0):
    wid = lax.axis_index("s") * NC + lax.axis_index("c")
    riota = lax.iota(jnp.int32, 16)
    zero16 = jnp.zeros((16,), jnp.float32)
    for s in range(2):
        for h in range(8):
            rows_v[s * SBUF + ZROW, pl.ds(h * 16, 16)] = zero16
    pltpu.sync_copy(tail_hbm, tail_v)
    for rt in range(16):
        for h in range(8):
            e0 = (h % 2) * 16
            vsub = h // 2
            v = plsc.load_gather(
                tail_v,
                [e0 + riota, jnp.full((16,), rt * 4 + vsub, jnp.int32)])
            cb = 4 * (rt & 3) + vsub
            for s in range(2):
                plsc.store_scatter(
                    rows_v.at[s * SBUF + TROW + rt],
                    [jnp.full((16,), vsub * 32, jnp.int32)
                     + ((e0 + riota) ^ cb)], v)

    def load_sup(c):
        sup = c // (CSUP // C)
        base = pl.multiple_of(wid * BPW + sup * CSUP, CSUP)
        for p in range(2):
            @pl.when(sup & 1 == p)
            def _(p=p):
                pltpu.sync_copy(idsT_hbm.at[wid * (BPW // CSUP) + sup],
                                ids2_v.at[pl.ds(p * G, G)])
                pltpu.sync_copy(lens_hbm.at[pl.ds(base, CSUP)],
                                lens_v.at[pl.ds(p * CSUP, CSUP)])

    def prep_and_fire(c, s):
        p = (c // (CSUP // C)) & 1
        cb32 = (c % (CSUP // C)) * C
        for g in range(G):
            for k in range(C // 16):
                idx_v[s * (G // 2) + g // 2, pl.ds((g % 2) * C + k * 16, 16)] \
                    = jnp.minimum(
                        ids2_v[p * G + g, pl.ds(cb32 + k * 16, 16)] >> 2,
                        NROW - 1)
        for g in range(G // 2):
            pltpu.async_copy(
                tableV_hbm.at[idx_v.at[s * (G // 2) + g]],
                rows_v.at[pl.ds(s * SBUF + g * 2 * C, 2 * C)], sem0)

    def drain(c, s):
        for g in range(G // 2):
            pltpu.make_async_copy(
                tableV_hbm.at[pl.ds(0, 2 * C)],
                rows_v.at[pl.ds(s * SBUF + g * 2 * C, 2 * C)], sem0).wait()

    def compute(c, s):
        p = (c // (CSUP // C)) & 1
        cb32 = (c % (CSUP // C)) * C

        def group_body(k, _):
            kb = k * 16
            lens16 = lens_v[pl.ds(p * CSUP + cb32 + kb, 16)]
            linv16 = 1.0 / lens16.astype(jnp.float32)
            acc = [zero16] * EMBED
            for j in range(G):
                ids16 = ids2_v[p * G + j, pl.ds(cb32 + kb, 16)]
                vsub = ids16 & (PACK - 1)
                cb = ((ids16 >> 2) & 3) * 4 + vsub
                colb = vsub * 32
                slotn = s * SBUF + j * C + kb + riota
                slott = s * SBUF + TROW + ((ids16 - VTAIL) >> 2)
                rowj = jnp.where(
                    lens16 > j,
                    jnp.where(ids16 >= VTAIL, slott, slotn),
                    s * SBUF + ZROW)
                for e in range(EMBED):
                    v = plsc.load_gather(rows_v, [rowj, colb + (cb ^ e)])
                    acc[e] = acc[e] + v
            for e in range(EMBED):
                outT_v[e, pl.ds(cb32 + kb, 16)] = acc[e] * linv16
            return 0

        lax.fori_loop(0, C // 16, group_body, 0)

    load_sup(0)
    prep_and_fire(0, 0)

    def pair_body(pi, _):
        for s in range(2):
            c = pi * 2 + s
            c1 = c + 1

            @pl.when(jnp.logical_and(c1 < NCHUNK, c1 % (CSUP // C) == 0))
            def _(c1=c1):
                load_sup(c1)

            drain(c, s)

            @pl.when(c1 < NCHUNK)
            def _(c1=c1, s=s):
                prep_and_fire(c1, 1 - s)
            compute(c, s)

            @pl.when(c % (CSUP // C) == CSUP // C - 1)
            def _(c=c):
                sup = c // (CSUP // C)
                pltpu.sync_copy(
                    outT_v,
                    outT_hbm.at[:, pl.ds(
                        pl.multiple_of(wid * BPW + sup * CSUP, CSUP), CSUP)])
        return 0

    lax.fori_loop(0, NCHUNK // 2, pair_body, 0)


@functools.partial(jax.jit, static_argnames=())
def kernel(subword_ids, lengths, table):
    tableT = table.T
    tail = lax.slice(tableT, (0, VTAIL), (EMBED, VOCAB))
    idsT = subword_ids.reshape(NW * BPW // CSUP, CSUP,
                               G).transpose(0, 2, 1)
    mesh = plsc.VectorSubcoreMesh(core_axis_name="c", subcore_axis_name="s")
    retile = pl.kernel(
        _retile_body,
        mesh=mesh,
        out_type=jax.ShapeDtypeStruct((NROW, 128), jnp.float32),
        scratch_types=[
            pltpu.VMEM((2, EMBED, 768), jnp.float32),
            pltpu.VMEM((2, 192, 128), jnp.float32),
            pltpu.SemaphoreType.DMA,
            pltpu.SemaphoreType.DMA,
            pltpu.SemaphoreType.DMA,
            pltpu.SemaphoreType.DMA,
        ],
        compiler_params=pltpu.CompilerParams(needs_layout_passes=False),
    )
    tableV = retile(tableT)
    lookup = pl.kernel(
        _lookup_body,
        mesh=mesh,
        out_type=jax.ShapeDtypeStruct((EMBED, B), jnp.float32),
        scratch_types=[
            pltpu.VMEM((2 * G, CSUP), jnp.int32),
            pltpu.VMEM((G, 2 * C), jnp.int32),
            pltpu.VMEM((2 * SBUF, 128), jnp.float32),
            pltpu.VMEM((2 * CSUP,), jnp.int32),
            pltpu.VMEM((EMBED, CSUP), jnp.float32),
            pltpu.VMEM((EMBED, 64), jnp.float32),
            pltpu.SemaphoreType.DMA,
        ],
        compiler_params=pltpu.CompilerParams(needs_layout_passes=False),
    )
    outT = lookup(tableV, idsT, lengths, tail)
    return outT.T
